# baseline (device time: 94464 ns/iter reference)
import jax
import jax.numpy as jnp
from jax import lax
from jax.experimental import pallas as pl
from jax.experimental.pallas import tpu as pltpu

N_Z = 4
E_LOCAL = 4
N_E = N_Z * E_LOCAL
CAP = 56
CHUNK = E_LOCAL * CAP
TQ = 512
XY_PEERS = ((0, 1), (1, 0), (1, 1))


def _body(x_ref, p_ref, w1_hbm, w2_hbm, out_ref, xs_ref, recv_ref,
          w1buf, w2buf, qsend, qrecv, loc_sems, fwd_send, fwd_recv,
          bwd_send, bwd_recv, xy_send, xy_recv, wsem1, wsem2):
    my_x = lax.axis_index("x")
    my_y = lax.axis_index("y")
    my_z = lax.axis_index("z")
    D = x_ref.shape[1]

    barrier_sem = pltpu.get_barrier_semaphore()
    for dz in range(1, N_Z):
        pl.semaphore_signal(
            barrier_sem, inc=1,
            device_id=(my_x, my_y, (my_z + dz) % N_Z),
            device_id_type=pl.DeviceIdType.MESH,
        )
    for ax, ay in XY_PEERS:
        pl.semaphore_signal(
            barrier_sem, inc=1,
            device_id=(my_x ^ ax, my_y ^ ay, my_z),
            device_id_type=pl.DeviceIdType.MESH,
        )
    pl.semaphore_wait(barrier_sem, N_Z - 1 + len(XY_PEERS))

    def load_w(hbm, j, buf, sem):
        cp = pltpu.make_async_copy(hbm.at[j], buf, sem)
        cp.start()
        return cp

    w1load = load_w(w1_hbm, 0, w1buf, wsem1)
    w2load = load_w(w2_hbm, 0, w2buf, wsem2)

    p_row = p_ref[...].reshape(1, TQ)
    xv = x_ref[...].astype(jnp.bfloat16)

    def gather_chunk(d):
        q = jax.lax.broadcasted_iota(jnp.int32, (CHUNK, TQ), 0) + d * CHUNK
        ohT = (q == p_row).astype(jnp.bfloat16)
        chunk = jax.lax.dot(ohT, xv, preferred_element_type=jnp.float32)
        xs_ref[pl.ds(E_LOCAL * d, E_LOCAL)] = (
            chunk.astype(jnp.bfloat16).reshape(E_LOCAL, CAP, D)
        )

    fwd = []
    for dz in range(1, N_Z):
        d = (my_z + dz) % N_Z
        gather_chunk(d)
        for j in range(E_LOCAL):
            rdma = pltpu.make_async_remote_copy(
                src_ref=xs_ref.at[pl.ds(E_LOCAL * d + j, 1)],
                dst_ref=recv_ref.at[j, pl.ds(dz, 1)],
                send_sem=fwd_send.at[j, dz],
                recv_sem=fwd_recv.at[j, dz],
                device_id=(my_x, my_y, d),
                device_id_type=pl.DeviceIdType.MESH,
            )
            rdma.start()
            fwd.append(rdma)
    gather_chunk(my_z)
    loc_fwd = []
    for j in range(E_LOCAL):
        cp = pltpu.make_async_copy(
            xs_ref.at[pl.ds(E_LOCAL * my_z + j, 1)],
            recv_ref.at[j, pl.ds(0, 1)],
            loc_sems.at[j],
        )
        cp.start()
        loc_fwd.append(cp)

    bwd = []
    for j in range(E_LOCAL):
        loc_fwd[j].wait()
        for b in range(1, N_Z):
            rcv = pltpu.make_async_remote_copy(
                src_ref=xs_ref.at[pl.ds(0, 1)],
                dst_ref=recv_ref.at[j, pl.ds(b, 1)],
                send_sem=fwd_send.at[j, b],
                recv_sem=fwd_recv.at[j, b],
                device_id=(my_x, my_y, my_z),
                device_id_type=pl.DeviceIdType.MESH,
            )
            rcv.wait_recv()
        a = recv_ref[j].reshape(N_Z * CAP, D).astype(jnp.float32)
        w1load.wait()
        h = jnp.maximum(
            jax.lax.dot(a, w1buf[...], preferred_element_type=jnp.float32), 0.0
        )
        if j + 1 < E_LOCAL:
            w1load = load_w(w1_hbm, j + 1, w1buf, wsem1)
        w2load.wait()
        o = jax.lax.dot(h, w2buf[...], preferred_element_type=jnp.float32)
        if j + 1 < E_LOCAL:
            w2load = load_w(w2_hbm, j + 1, w2buf, wsem2)
        recv_ref[j] = o.astype(jnp.bfloat16).reshape(N_Z, CAP, D)

        cp = pltpu.make_async_copy(
            recv_ref.at[j, pl.ds(0, 1)],
            xs_ref.at[pl.ds(E_LOCAL * my_z + j, 1)],
            loc_sems.at[j],
        )
        cp.start()
        loc_fwd[j] = cp
        for b in range(1, N_Z):
            s = (my_z - b) % N_Z
            rdma = pltpu.make_async_remote_copy(
                src_ref=recv_ref.at[j, pl.ds(b, 1)],
                dst_ref=xs_ref.at[pl.ds(E_LOCAL * my_z + j, 1)],
                send_sem=bwd_send.at[j, b],
                recv_sem=bwd_recv.at[j, b],
                device_id=(my_x, my_y, s),
                device_id_type=pl.DeviceIdType.MESH,
            )
            rdma.start()
            bwd.append(rdma)

    p_col = p_ref[...].reshape(TQ, 1)

    def scatter_chunk(d, acc):
        q = jax.lax.broadcasted_iota(jnp.int32, (TQ, CHUNK), 1) + d * CHUNK
        oh = (q == p_col).astype(jnp.bfloat16)
        res = xs_ref[pl.ds(E_LOCAL * d, E_LOCAL)].reshape(CHUNK, D)
        contrib = jax.lax.dot(oh, res, preferred_element_type=jnp.float32)
        return contrib if acc is None else acc + contrib

    for j in range(E_LOCAL):
        loc_fwd[j].wait()
    acc = scatter_chunk(my_z, None)
    for f in range(1, N_Z):
        d = (my_z + f) % N_Z
        for j in range(E_LOCAL):
            rcv = pltpu.make_async_remote_copy(
                src_ref=recv_ref.at[j, pl.ds(0, 1)],
                dst_ref=xs_ref.at[pl.ds(E_LOCAL * d + j, 1)],
                send_sem=bwd_send.at[j, f],
                recv_sem=bwd_recv.at[j, f],
                device_id=(my_x, my_y, my_z),
                device_id_type=pl.DeviceIdType.MESH,
            )
            rcv.wait_recv()
        acc = scatter_chunk(d, acc)

    my_q = 2 * my_x + my_y
    out_ref[pl.ds(my_q * TQ, TQ), :] = acc
    qsend[0] = acc.astype(jnp.bfloat16)

    xy = []
    for t, (ax, ay) in enumerate(XY_PEERS):
        rdma = pltpu.make_async_remote_copy(
            src_ref=qsend.at[pl.ds(0, 1)],
            dst_ref=qrecv.at[pl.ds(t, 1)],
            send_sem=xy_send.at[t],
            recv_sem=xy_recv.at[t],
            device_id=(my_x ^ ax, my_y ^ ay, my_z),
            device_id_type=pl.DeviceIdType.MESH,
        )
        rdma.start()
        xy.append(rdma)
    for t, (ax, ay) in enumerate(XY_PEERS):
        xy[t].wait_recv()
        peer_q = 2 * (my_x ^ ax) + (my_y ^ ay)
        out_ref[pl.ds(peer_q * TQ, TQ), :] = qrecv[t].astype(jnp.float32)

    for rdma in fwd:
        rdma.wait_send()
    for rdma in bwd:
        rdma.wait_send()
    for rdma in xy:
        rdma.wait_send()


def kernel(x, assign, W1, W2):
    T, D = x.shape

    my_q = 2 * lax.axis_index("x") + lax.axis_index("y")
    x_q = lax.dynamic_slice(x, (my_q * TQ, 0), (TQ, D))
    a_q = lax.dynamic_slice(assign, (my_q * TQ,), (TQ,))

    onehot = a_q[:, None] == jnp.arange(N_E, dtype=a_q.dtype)[None, :]
    cum = jnp.cumsum(onehot.astype(jnp.int32), axis=0)
    rank = jnp.sum(jnp.where(onehot, cum - 1, 0), axis=1)
    p = a_q * CAP + rank

    return pl.pallas_call(
        _body,
        out_shape=jax.ShapeDtypeStruct((T, D), jnp.float32),
        in_specs=[
            pl.BlockSpec(memory_space=pltpu.VMEM),
            pl.BlockSpec(memory_space=pltpu.VMEM),
            pl.BlockSpec(memory_space=pl.ANY),
            pl.BlockSpec(memory_space=pl.ANY),
        ],
        out_specs=pl.BlockSpec(memory_space=pltpu.VMEM),
        scratch_shapes=[
            pltpu.VMEM((N_E, CAP, D), jnp.bfloat16),
            pltpu.VMEM((E_LOCAL, N_Z, CAP, D), jnp.bfloat16),
            pltpu.VMEM(W1.shape[1:], jnp.float32),
            pltpu.VMEM(W2.shape[1:], jnp.float32),
            pltpu.VMEM((1, TQ, D), jnp.bfloat16),
            pltpu.VMEM((len(XY_PEERS), TQ, D), jnp.bfloat16),
            pltpu.SemaphoreType.DMA((E_LOCAL,)),
            pltpu.SemaphoreType.DMA((E_LOCAL, N_Z)),
            pltpu.SemaphoreType.DMA((E_LOCAL, N_Z)),
            pltpu.SemaphoreType.DMA((E_LOCAL, N_Z)),
            pltpu.SemaphoreType.DMA((E_LOCAL, N_Z)),
            pltpu.SemaphoreType.DMA((len(XY_PEERS),)),
            pltpu.SemaphoreType.DMA((len(XY_PEERS),)),
            pltpu.SemaphoreType.DMA,
            pltpu.SemaphoreType.DMA,
        ],
        compiler_params=pltpu.CompilerParams(
            collective_id=0, vmem_limit_bytes=100 * 1024 * 1024
        ),
    )(x_q, p.astype(jnp.int32), W1, W2)
